# baseline (speedup 1.0000x reference)
import functools

import jax
import jax.numpy as jnp
from jax import lax
from jax.experimental import pallas as pl
from jax.experimental.pallas import tpu as pltpu
from jax.experimental.pallas import tpu_sc as plsc

HIDDEN = 64
VOCAB = 2016
SCALER = 7

NC = 2
NS = 16
NW = NC * NS
L = 16

TOTAL = 4096 * 200
PER_W = TOTAL // NW
BLK = 128
NBLK = PER_W // BLK


def _table_body(emb_ref, w_ref, b_ref, out_ref):
    t = lax.dot_general(
        emb_ref[:], w_ref[:], (((1,), (1,)), ((), ())),
        preferred_element_type=jnp.float32,
    )
    t = t + b_ref[:]
    out_ref[:] = jnp.concatenate([t[:, 1:], jnp.sin(t[:, :1])], axis=1)


def _build_table(emb, W, b):
    return pl.pallas_call(
        _table_body,
        out_shape=jax.ShapeDtypeStruct((VOCAB, HIDDEN), jnp.float32),
    )(emb, W, b.reshape(1, HIDDEN))


@functools.partial(
    pl.kernel,
    mesh=plsc.VectorSubcoreMesh(core_axis_name="c", subcore_axis_name="s"),
    compiler_params=pltpu.CompilerParams(use_tc_tiling_on_sc=False),
    out_type=jax.ShapeDtypeStruct((TOTAL, HIDDEN), jnp.float32),
    scratch_types=[
        pltpu.VMEM((BLK,), jnp.int32),
        pltpu.VMEM((BLK,), jnp.int32),
        pltpu.VMEM((1, BLK), jnp.int32),
        pltpu.VMEM((BLK, HIDDEN), jnp.float32),
        pltpu.SemaphoreType.DMA,
    ],
)
def _sc_gather(tod_hbm, dow_hbm, table_hbm, out_hbm,
               tod_v, dow_v, idx_v, rows_v, gsem):
    wid = lax.axis_index("s") * NC + lax.axis_index("c")
    base = wid * PER_W

    def body(g, carry):
        off = base + g * BLK
        pltpu.sync_copy(tod_hbm.at[pl.ds(off, BLK)], tod_v)
        pltpu.sync_copy(dow_hbm.at[pl.ds(off, BLK)], dow_v)
        for i in range(BLK // L):
            s = pl.ds(i * L, L)
            idx_v[0, s] = tod_v[s] * SCALER + dow_v[s]
        pltpu.async_copy(table_hbm.at[idx_v.at[0]], rows_v, gsem).wait()
        pltpu.sync_copy(rows_v, out_hbm.at[pl.ds(off, BLK)])
        return carry

    lax.fori_loop(0, NBLK, body, 0)


def kernel(tod, dow, emb, W, b):
    table = _build_table(emb, W, b)
    out = _sc_gather(tod.reshape(-1), dow.reshape(-1), table)
    return out.reshape(tod.shape + (HIDDEN,))


# speedup vs baseline: 1.1964x; 1.1964x over previous
import functools

import jax
import jax.numpy as jnp
from jax import lax
from jax.experimental import pallas as pl
from jax.experimental.pallas import tpu as pltpu
from jax.experimental.pallas import tpu_sc as plsc

HIDDEN = 64
VOCAB = 2016
SCALER = 7

NC = 2
NS = 16
NW = NC * NS
L = 16

TOTAL = 4096 * 200
PER_W = TOTAL // NW
BLK = 128
NBLK = PER_W // BLK


def _table_body(emb_ref, w_ref, b_ref, out_ref):
    t = lax.dot_general(
        emb_ref[:], w_ref[:], (((1,), (1,)), ((), ())),
        preferred_element_type=jnp.float32,
    )
    t = t + b_ref[:]
    out_ref[:] = jnp.concatenate([t[:, 1:], jnp.sin(t[:, :1])], axis=1)


def _build_table(emb, W, b):
    return pl.pallas_call(
        _table_body,
        out_shape=jax.ShapeDtypeStruct((VOCAB, HIDDEN), jnp.float32),
    )(emb, W, b.reshape(1, HIDDEN))


@functools.partial(
    pl.kernel,
    mesh=plsc.VectorSubcoreMesh(core_axis_name="c", subcore_axis_name="s"),
    compiler_params=pltpu.CompilerParams(use_tc_tiling_on_sc=False),
    out_type=jax.ShapeDtypeStruct((TOTAL, HIDDEN), jnp.float32),
    scratch_types=[
        pltpu.VMEM((PER_W,), jnp.int32),
        pltpu.VMEM((PER_W,), jnp.int32),
        pltpu.VMEM((1, BLK), jnp.int32),
        pltpu.VMEM((BLK, HIDDEN), jnp.float32),
        pltpu.SemaphoreType.DMA,
        pltpu.SemaphoreType.DMA,
    ],
)
def _sc_gather(tod_hbm, dow_hbm, table_hbm, out_hbm,
               tod_v, dow_v, idx_v, rows_v, ssem, gsem):
    wid = lax.axis_index("s") * NC + lax.axis_index("c")
    base = wid * PER_W

    st = pltpu.async_copy(tod_hbm.at[pl.ds(base, PER_W)], tod_v, ssem)
    sd = pltpu.async_copy(dow_hbm.at[pl.ds(base, PER_W)], dow_v, ssem)
    st.wait()
    sd.wait()

    def body(g, carry):
        goff = g * BLK
        for i in range(BLK // L):
            s = pl.ds(goff + i * L, L)
            idx_v[0, pl.ds(i * L, L)] = tod_v[s] * SCALER + dow_v[s]
        pltpu.async_copy(table_hbm.at[idx_v.at[0]], rows_v, gsem).wait()
        pltpu.sync_copy(rows_v, out_hbm.at[pl.ds(base + goff, BLK)])
        return carry

    lax.fori_loop(0, NBLK, body, 0)


def kernel(tod, dow, emb, W, b):
    table = _build_table(emb, W, b)
    out = _sc_gather(tod.reshape(-1), dow.reshape(-1), table)
    return out.reshape(tod.shape + (HIDDEN,))


# speedup vs baseline: 1.2710x; 1.0623x over previous
import functools

import jax
import jax.numpy as jnp
from jax import lax
from jax.experimental import pallas as pl
from jax.experimental.pallas import tpu as pltpu
from jax.experimental.pallas import tpu_sc as plsc

HIDDEN = 64
VOCAB = 2016
SCALER = 7

NC = 2
NS = 16
NW = NC * NS
L = 16

TOTAL = 4096 * 200
PER_W = TOTAL // NW
BLK = 128
NBLK = PER_W // BLK


def _table_body(emb_ref, w_ref, b_ref, out_ref):
    t = lax.dot_general(
        emb_ref[:], w_ref[:], (((1,), (1,)), ((), ())),
        preferred_element_type=jnp.float32,
    )
    t = t + b_ref[:]
    out_ref[:] = jnp.concatenate([t[:, 1:], jnp.sin(t[:, :1])], axis=1)


def _build_table(emb, W, b):
    return pl.pallas_call(
        _table_body,
        out_shape=jax.ShapeDtypeStruct((VOCAB, HIDDEN), jnp.float32),
    )(emb, W, b.reshape(1, HIDDEN))


@functools.partial(
    pl.kernel,
    mesh=plsc.VectorSubcoreMesh(core_axis_name="c", subcore_axis_name="s"),
    compiler_params=pltpu.CompilerParams(use_tc_tiling_on_sc=False),
    out_type=jax.ShapeDtypeStruct((TOTAL, HIDDEN), jnp.float32),
    scratch_types=[
        pltpu.VMEM((PER_W,), jnp.int32),
        pltpu.VMEM((PER_W,), jnp.int32),
        pltpu.VMEM((2, BLK), jnp.int32),
        pltpu.VMEM((2, BLK, HIDDEN), jnp.float32),
        pltpu.SemaphoreType.DMA,
        pltpu.SemaphoreType.DMA,
        pltpu.SemaphoreType.DMA,
        pltpu.SemaphoreType.DMA,
        pltpu.SemaphoreType.DMA,
    ],
)
def _sc_gather(tod_hbm, dow_hbm, table_hbm, out_hbm,
               tod_v, dow_v, idx_v, rows_v, ssem, gsem0, gsem1, wsem0, wsem1):
    wid = lax.axis_index("s") * NC + lax.axis_index("c")
    base = wid * PER_W
    gsem = (gsem0, gsem1)
    wsem = (wsem0, wsem1)

    st = pltpu.async_copy(tod_hbm.at[pl.ds(base, PER_W)], tod_v, ssem)
    sd = pltpu.async_copy(dow_hbm.at[pl.ds(base, PER_W)], dow_v, ssem)
    st.wait()
    sd.wait()

    def calc_idx(goff, b):
        for i in range(BLK // L):
            s = pl.ds(goff + i * L, L)
            idx_v[b, pl.ds(i * L, L)] = tod_v[s] * SCALER + dow_v[s]

    def start_gather(b):
        return pltpu.async_copy(table_hbm.at[idx_v.at[b]], rows_v.at[b],
                                gsem[b])

    def wait_gather(b):
        pltpu.make_async_copy(table_hbm.at[idx_v.at[b]], rows_v.at[b],
                              gsem[b]).wait()

    def start_write(goff, b):
        return pltpu.async_copy(rows_v.at[b],
                                out_hbm.at[pl.ds(base + goff, BLK)], wsem[b])

    def wait_write(b):
        pltpu.make_async_copy(rows_v.at[b],
                              out_hbm.at[pl.ds(base, BLK)], wsem[b]).wait()

    calc_idx(0, 0)
    start_gather(0)

    def body(p, carry):
        g0 = p * 2 * BLK
        g1 = g0 + BLK
        wait_gather(0)
        start_write(g0, 0)
        calc_idx(g1, 1)

        @pl.when(p > 0)
        def _():
            wait_write(1)
        start_gather(1)
        wait_gather(1)
        start_write(g1, 1)

        @pl.when(p + 1 < NBLK // 2)
        def _():
            calc_idx(g0 + 2 * BLK, 0)
            wait_write(0)
            start_gather(0)
        return carry

    lax.fori_loop(0, NBLK // 2, body, 0)
    wait_write(0)
    wait_write(1)


def kernel(tod, dow, emb, W, b):
    table = _build_table(emb, W, b)
    out = _sc_gather(tod.reshape(-1), dow.reshape(-1), table)
    return out.reshape(tod.shape + (HIDDEN,))


# speedup vs baseline: 1.3592x; 1.0693x over previous
import functools

import jax
import jax.numpy as jnp
from jax import lax
from jax.experimental import pallas as pl
from jax.experimental.pallas import tpu as pltpu
from jax.experimental.pallas import tpu_sc as plsc

HIDDEN = 64
VOCAB = 2016
SCALER = 7

NC = 2
NS = 16
NW = NC * NS
L = 16

TOTAL = 4096 * 200
PER_W = TOTAL // NW
BLK = 128
SB = 4
SBLK = SB * BLK
NSB = PER_W // SBLK
NBLK_TOTAL = TOTAL // BLK


def _table_body(emb_ref, w_ref, b_ref, out_ref):
    t = lax.dot_general(
        emb_ref[:], w_ref[:], (((1,), (1,)), ((), ())),
        preferred_element_type=jnp.float32,
    )
    t = t + b_ref[:]
    out_ref[:] = jnp.concatenate([t[:, 1:], jnp.sin(t[:, :1])], axis=1)


def _build_table(emb, W, b):
    return pl.pallas_call(
        _table_body,
        out_shape=jax.ShapeDtypeStruct((VOCAB, HIDDEN), jnp.float32),
    )(emb, W, b.reshape(1, HIDDEN))


@functools.partial(
    pl.kernel,
    mesh=plsc.VectorSubcoreMesh(core_axis_name="c", subcore_axis_name="s"),
    compiler_params=pltpu.CompilerParams(use_tc_tiling_on_sc=False),
    out_type=jax.ShapeDtypeStruct((NBLK_TOTAL, BLK, HIDDEN), jnp.float32),
    scratch_types=[
        pltpu.VMEM((PER_W,), jnp.int32),
        pltpu.VMEM((PER_W,), jnp.int32),
        pltpu.VMEM((2, SB, BLK), jnp.int32),
        pltpu.VMEM((2, SB, BLK, HIDDEN), jnp.float32),
        pltpu.SemaphoreType.DMA,
        pltpu.SemaphoreType.DMA,
        pltpu.SemaphoreType.DMA,
        pltpu.SemaphoreType.DMA,
        pltpu.SemaphoreType.DMA,
    ],
)
def _sc_gather(tod_hbm, dow_hbm, table_hbm, out_hbm,
               tod_v, dow_v, idx_v, rows_v, ssem, gsem0, gsem1, wsem0, wsem1):
    wid = lax.axis_index("s") * NC + lax.axis_index("c")
    base = wid * PER_W
    bblk = wid * (PER_W // BLK)
    gsem = (gsem0, gsem1)
    wsem = (wsem0, wsem1)

    st = pltpu.async_copy(tod_hbm.at[pl.ds(base, PER_W)], tod_v, ssem)
    sd = pltpu.async_copy(dow_hbm.at[pl.ds(base, PER_W)], dow_v, ssem)
    st.wait()
    sd.wait()

    def calc_idx(soff, b):
        for j in range(SB):
            for i in range(BLK // L):
                s = pl.ds(soff + j * BLK + i * L, L)
                idx_v[b, j, pl.ds(i * L, L)] = tod_v[s] * SCALER + dow_v[s]

    def start_gathers(b):
        for j in range(SB):
            pltpu.async_copy(table_hbm.at[idx_v.at[b, j]], rows_v.at[b, j],
                             gsem[b])

    def wait_gathers(b):
        for j in range(SB):
            pltpu.make_async_copy(table_hbm.at[idx_v.at[b, j]],
                                  rows_v.at[b, j], gsem[b]).wait()

    def start_write(sb, b):
        pltpu.async_copy(rows_v.at[b], out_hbm.at[pl.ds(bblk + sb * SB, SB)],
                         wsem[b])

    def wait_write(b):
        pltpu.make_async_copy(rows_v.at[b], out_hbm.at[pl.ds(bblk, SB)],
                              wsem[b]).wait()

    calc_idx(0, 0)
    start_gathers(0)

    def body(p, carry):
        s0 = p * 2
        s1 = s0 + 1
        wait_gathers(0)
        start_write(s0, 0)
        calc_idx(s1 * SBLK, 1)

        @pl.when(p > 0)
        def _():
            wait_write(1)
        start_gathers(1)
        wait_gathers(1)
        start_write(s1, 1)

        @pl.when(p + 1 < NSB // 2)
        def _():
            calc_idx((s0 + 2) * SBLK, 0)
            wait_write(0)
            start_gathers(0)
        return carry

    lax.fori_loop(0, NSB // 2, body, 0)
    wait_write(0)
    wait_write(1)


def kernel(tod, dow, emb, W, b):
    table = _build_table(emb, W, b)
    out = _sc_gather(tod.reshape(-1), dow.reshape(-1), table)
    return out.reshape(tod.shape + (HIDDEN,))


# speedup vs baseline: 1.6451x; 1.2104x over previous
import functools

import jax
import jax.numpy as jnp
from jax import lax
from jax.experimental import pallas as pl
from jax.experimental.pallas import tpu as pltpu
from jax.experimental.pallas import tpu_sc as plsc

HIDDEN = 64
VOCAB = 2016
SCALER = 7

NC = 2
NS = 16
NW = NC * NS
L = 16

TOTAL = 4096 * 200
PER_W = TOTAL // NW
BLK = 128
SB = 4
SBLK = SB * BLK
NSB = PER_W // SBLK
NBLK_TOTAL = TOTAL // BLK


def _table_body(emb_ref, w_ref, b_ref, out_ref):
    t = lax.dot_general(
        emb_ref[:], w_ref[:], (((1,), (1,)), ((), ())),
        preferred_element_type=jnp.float32,
    )
    t = t + b_ref[:]
    out_ref[:] = jnp.concatenate([t[:, 1:], jnp.sin(t[:, :1])], axis=1)


def _build_table(emb, W, b):
    return pl.pallas_call(
        _table_body,
        out_shape=jax.ShapeDtypeStruct((VOCAB, HIDDEN), jnp.float32),
    )(emb, W, b.reshape(1, HIDDEN))


@functools.partial(
    pl.kernel,
    mesh=plsc.VectorSubcoreMesh(core_axis_name="c", subcore_axis_name="s"),
    compiler_params=pltpu.CompilerParams(use_tc_tiling_on_sc=False),
    out_type=jax.ShapeDtypeStruct((NBLK_TOTAL, BLK, HIDDEN), jnp.float32),
    scratch_types=[
        pltpu.VMEM((PER_W,), jnp.int32),
        pltpu.VMEM((PER_W,), jnp.int32),
        pltpu.VMEM((2, SB, BLK), jnp.int32),
        pltpu.VMEM((2, SB, BLK, HIDDEN), jnp.float32),
        pltpu.VMEM_SHARED((VOCAB, HIDDEN), jnp.float32),
        pltpu.SemaphoreType.DMA,
        pltpu.SemaphoreType.DMA,
        pltpu.SemaphoreType.DMA,
        pltpu.SemaphoreType.DMA,
        pltpu.SemaphoreType.DMA,
    ],
)
def _sc_gather(tod_hbm, dow_hbm, table_hbm, out_hbm,
               tod_v, dow_v, idx_v, rows_v, table_sh,
               ssem, gsem0, gsem1, wsem0, wsem1):
    sid = lax.axis_index("s")
    wid = sid * NC + lax.axis_index("c")
    base = wid * PER_W
    bblk = wid * (PER_W // BLK)
    gsem = (gsem0, gsem1)
    wsem = (wsem0, wsem1)

    st = pltpu.async_copy(tod_hbm.at[pl.ds(base, PER_W)], tod_v, ssem)
    sd = pltpu.async_copy(dow_hbm.at[pl.ds(base, PER_W)], dow_v, ssem)

    @pl.when(sid == 0)
    def _():
        pltpu.sync_copy(table_hbm, table_sh)

    st.wait()
    sd.wait()
    plsc.subcore_barrier()

    def calc_idx(soff, b):
        for j in range(SB):
            for i in range(BLK // L):
                s = pl.ds(soff + j * BLK + i * L, L)
                idx_v[b, j, pl.ds(i * L, L)] = tod_v[s] * SCALER + dow_v[s]

    def start_gathers(b):
        for j in range(SB):
            pltpu.async_copy(table_sh.at[idx_v.at[b, j]], rows_v.at[b, j],
                             gsem[b])

    def wait_gathers(b):
        for j in range(SB):
            pltpu.make_async_copy(table_sh.at[idx_v.at[b, j]],
                                  rows_v.at[b, j], gsem[b]).wait()

    def start_write(sb, b):
        pltpu.async_copy(rows_v.at[b], out_hbm.at[pl.ds(bblk + sb * SB, SB)],
                         wsem[b])

    def wait_write(b):
        pltpu.make_async_copy(rows_v.at[b], out_hbm.at[pl.ds(bblk, SB)],
                              wsem[b]).wait()

    calc_idx(0, 0)
    start_gathers(0)

    def body(p, carry):
        s0 = p * 2
        s1 = s0 + 1
        wait_gathers(0)
        start_write(s0, 0)
        calc_idx(s1 * SBLK, 1)

        @pl.when(p > 0)
        def _():
            wait_write(1)
        start_gathers(1)
        wait_gathers(1)
        start_write(s1, 1)

        @pl.when(p + 1 < NSB // 2)
        def _():
            calc_idx((s0 + 2) * SBLK, 0)
            wait_write(0)
            start_gathers(0)
        return carry

    lax.fori_loop(0, NSB // 2, body, 0)
    wait_write(0)
    wait_write(1)


def kernel(tod, dow, emb, W, b):
    table = _build_table(emb, W, b)
    out = _sc_gather(tod.reshape(-1), dow.reshape(-1), table)
    return out.reshape(tod.shape + (HIDDEN,))
